# baseline (speedup 1.0000x reference)
import functools

import jax
import jax.numpy as jnp
from jax import lax
from jax.experimental import pallas as pl
from jax.experimental.pallas import tpu as pltpu
from jax.experimental.pallas import tpu_sc as plsc

_SC_CORES = 2
_SC_SUBCORES = 16
_NW = _SC_CORES * _SC_SUBCORES
_CH = 128
_WC = 128
_EB = 2048
_BN = 1000
_BE = 2048

_F32 = jnp.float32


def _ln(x, g, b):
    mu = jnp.mean(x, axis=-1, keepdims=True)
    var = jnp.mean((x - mu) ** 2, axis=-1, keepdims=True)
    return g * (x - mu) / jnp.sqrt(var + 1e-5) + b


def _silu(x):
    return x * jax.nn.sigmoid(x)


def _pack_bf16(x):
    k = x.shape[-1] // 2
    a = x[:, :k].astype(jnp.bfloat16)
    b = x[:, k:].astype(jnp.bfloat16)
    au = lax.bitcast_convert_type(a, jnp.uint16).astype(jnp.uint32)
    bu = lax.bitcast_convert_type(b, jnp.uint16).astype(jnp.uint32)
    return lax.bitcast_convert_type((au << 16) | bu, _F32)


def _unpack_bf16(p):
    u = lax.bitcast_convert_type(p, jnp.uint32)
    hi = lax.bitcast_convert_type((u >> 16).astype(jnp.uint16), jnp.bfloat16)
    lo = lax.bitcast_convert_type(u.astype(jnp.uint16), jnp.bfloat16)
    return jnp.concatenate([hi, lo], axis=-1)


def _sc_gather(table, idx):
    ew = idx.shape[0]
    tail = table.shape[1:]
    per_w = ew // _NW
    nring = 4
    n4 = per_w // (nring * _CH)
    mesh = plsc.VectorSubcoreMesh(core_axis_name="c", subcore_axis_name="s")

    @functools.partial(
        pl.kernel,
        out_type=jax.ShapeDtypeStruct((ew,) + tail, table.dtype),
        mesh=mesh,
        scratch_types=[
            pltpu.VMEM((per_w,), jnp.int32),
        ] + [pltpu.VMEM((_CH,) + tail, table.dtype)] * nring
          + [pltpu.SemaphoreType.DMA] * (2 * nring),
    )
    def gk(table_hbm, idx_hbm, out_hbm, idx_v, *bufs_sems):
        rows = bufs_sems[:nring]
        sems = bufs_sems[nring:2 * nring]
        wsems = bufs_sems[2 * nring:]
        wid = lax.axis_index("s") * _SC_CORES + lax.axis_index("c")
        base = wid * per_w
        pltpu.sync_copy(idx_hbm.at[pl.ds(base, per_w)], idx_v)

        def body(p, carry):
            offs = [(nring * p + j) * _CH for j in range(nring)]
            descs = []
            for j in range(nring):
                @pl.when(p > 0)
                def _(j=j):
                    pltpu.make_async_copy(
                        rows[j], out_hbm.at[pl.ds(base + offs[j], _CH)],
                        wsems[j]).wait()
                descs.append(pltpu.async_copy(
                    table_hbm.at[idx_v.at[pl.ds(offs[j], _CH)]],
                    rows[j], sems[j]))
            for j in range(nring):
                descs[j].wait()
                pltpu.async_copy(
                    rows[j], out_hbm.at[pl.ds(base + offs[j], _CH)],
                    wsems[j])
            return carry

        lax.fori_loop(0, n4, body, 0)
        for j in range(nring):
            pltpu.make_async_copy(
                rows[j], out_hbm.at[pl.ds(base, _CH)], wsems[j]).wait()

    return gk(table, idx)


def _sc_gather2(table, idx_a, idx_b):
    ew = idx_a.shape[0]
    d = table.shape[1]
    per_w = ew // _NW
    n_ch = per_w // _CH
    mesh = plsc.VectorSubcoreMesh(core_axis_name="c", subcore_axis_name="s")

    @functools.partial(
        pl.kernel,
        out_type=[
            jax.ShapeDtypeStruct((ew, d), _F32),
            jax.ShapeDtypeStruct((ew, d), _F32),
        ],
        mesh=mesh,
        scratch_types=[
            pltpu.VMEM((per_w,), jnp.int32),
            pltpu.VMEM((per_w,), jnp.int32),
            pltpu.VMEM((_CH, d), _F32),
            pltpu.VMEM((_CH, d), _F32),
            pltpu.SemaphoreType.DMA,
            pltpu.SemaphoreType.DMA,
            pltpu.SemaphoreType.DMA,
            pltpu.SemaphoreType.DMA,
        ],
    )
    def gk(table_hbm, ia_hbm, ib_hbm, oa_hbm, ob_hbm,
           ia_v, ib_v, rows0, rows1, sem0, sem1, wsem0, wsem1):
        wid = lax.axis_index("s") * _SC_CORES + lax.axis_index("c")
        base = wid * per_w
        pltpu.sync_copy(ia_hbm.at[pl.ds(base, per_w)], ia_v)
        pltpu.sync_copy(ib_hbm.at[pl.ds(base, per_w)], ib_v)

        def body(t, carry):
            off = t * _CH

            @pl.when(t > 0)
            def _():
                pltpu.make_async_copy(
                    rows0, oa_hbm.at[pl.ds(base + off, _CH)], wsem0).wait()
                pltpu.make_async_copy(
                    rows1, ob_hbm.at[pl.ds(base + off, _CH)], wsem1).wait()

            d0 = pltpu.async_copy(
                table_hbm.at[ia_v.at[pl.ds(off, _CH)]], rows0, sem0)
            d1 = pltpu.async_copy(
                table_hbm.at[ib_v.at[pl.ds(off, _CH)]], rows1, sem1)
            d0.wait()
            pltpu.async_copy(rows0, oa_hbm.at[pl.ds(base + off, _CH)], wsem0)
            d1.wait()
            pltpu.async_copy(rows1, ob_hbm.at[pl.ds(base + off, _CH)], wsem1)
            return carry

        lax.fori_loop(0, n_ch, body, 0)
        pltpu.make_async_copy(
            rows0, oa_hbm.at[pl.ds(base, _CH)], wsem0).wait()
        pltpu.make_async_copy(
            rows1, ob_hbm.at[pl.ds(base, _CH)], wsem1).wait()

    return gk(table, idx_a, idx_b)


def _embed_body(af_ref, w_ref, b_ref, g_ref, be_ref, wm_ref, h_ref, m_ref):
    x = jnp.dot(af_ref[...], w_ref[...], preferred_element_type=_F32) + b_ref[...]
    h = _ln(_silu(x), g_ref[...], be_ref[...])
    h_ref[...] = h
    m_ref[...] = _pack_bf16(
        jnp.dot(h, wm_ref[...], preferred_element_type=_F32))


def _embed_call(af, w, b, g, be, wm):
    n = af.shape[0]
    gh = w.shape[1]
    grid = (n // _BN,)
    full = lambda i: (0, 0)
    return pl.pallas_call(
        _embed_body,
        grid=grid,
        in_specs=[
            pl.BlockSpec((_BN, af.shape[1]), lambda i: (i, 0)),
            pl.BlockSpec(w.shape, full),
            pl.BlockSpec(b.shape, full),
            pl.BlockSpec(g.shape, full),
            pl.BlockSpec(be.shape, full),
            pl.BlockSpec(wm.shape, full),
        ],
        out_specs=[
            pl.BlockSpec((_BN, gh), lambda i: (i, 0)),
            pl.BlockSpec((_BN, gh // 2), lambda i: (i, 0)),
        ],
        out_shape=[
            jax.ShapeDtypeStruct((n, gh), _F32),
            jax.ShapeDtypeStruct((n, gh // 2), _F32),
        ],
    )(af, w, b, g, be, wm)


def _enc_body(asrc_ref, adst_ref, d_ref, c_ref, w1a_ref, w1b_ref, w1r_ref,
              b1_ref, w2_ref, b2_ref, g_ref, be_ref, out_ref, *, wsq):
    bf = jnp.bfloat16
    dd = d_ref[...]
    nb = dd.shape[0]
    bins = c_ref.shape[1]
    dist3 = dd[:, :, None]
    cen3 = c_ref[...].reshape(1, 1, bins)
    rbf = jnp.exp(-((dist3 - cen3) ** 2) / wsq).reshape(nb * 128, bins)
    x = (jnp.dot(_unpack_bf16(asrc_ref[...]), w1a_ref[...].astype(bf),
                 preferred_element_type=_F32)
         + jnp.dot(_unpack_bf16(adst_ref[...]), w1b_ref[...].astype(bf),
                   preferred_element_type=_F32)
         + jnp.dot(rbf, w1r_ref[...], preferred_element_type=_F32)
         + b1_ref[...])
    x = _silu(x)
    y = _silu(jnp.dot(x.astype(bf), w2_ref[...].astype(bf),
                      preferred_element_type=_F32) + b2_ref[...])
    out_ref[...] = _ln(y, g_ref[...], be_ref[...])


def _enc_call(asrc, adst, d2d, centers, w1a, w1b, w1r, b1, w2, b2, g, be, wsq):
    ew = asrc.shape[0]
    eh = w2.shape[1]
    grid = (ew // _BE,)
    full = lambda i: (0, 0)
    return pl.pallas_call(
        functools.partial(_enc_body, wsq=wsq),
        grid=grid,
        in_specs=[
            pl.BlockSpec((_BE, asrc.shape[1]), lambda i: (i, 0)),
            pl.BlockSpec((_BE, adst.shape[1]), lambda i: (i, 0)),
            pl.BlockSpec((_BE // 128, 128), lambda i: (i, 0)),
            pl.BlockSpec(centers.shape, full),
            pl.BlockSpec(w1a.shape, full),
            pl.BlockSpec(w1b.shape, full),
            pl.BlockSpec(w1r.shape, full),
            pl.BlockSpec(b1.shape, full),
            pl.BlockSpec(w2.shape, full),
            pl.BlockSpec(b2.shape, full),
            pl.BlockSpec(g.shape, full),
            pl.BlockSpec(be.shape, full),
        ],
        out_specs=pl.BlockSpec((_BE, eh), lambda i: (i, 0)),
        out_shape=jax.ShapeDtypeStruct((ew, eh), _F32),
    )(asrc, adst, d2d, centers, w1a, w1b, w1r, b1, w2, b2, g, be)


def _seg_body(starts_ref, ef_hbm, m_hbm, dr_hbm, wg_ref, bg_ref,
              h_ref, ws_ref, bs_ref, g_ref, b_ref, wm_ref,
              h2_ref, m2_ref, hbf_ref,
              acc, ef_a, m_a, dr_a, ef_b, m_b, dr_b,
              sa1, sa2, sa3, sb1, sb2, sb3, *, eh, gh):
    c = pl.program_id(0)
    s = starts_ref[c]
    e = starts_ref[c + 1]
    base0 = (s // 128) * 128
    n_it = (e - base0 + _EB - 1) // _EB
    acc[...] = jnp.zeros_like(acc)
    wg = wg_ref[...].astype(jnp.bfloat16)
    bg = bg_ref[...]

    def copies(t, efb, mb, drb, s1, s2, s3):
        b = base0 + t * _EB
        return (pltpu.make_async_copy(ef_hbm.at[pl.ds(b, _EB)], efb, s1),
                pltpu.make_async_copy(m_hbm.at[pl.ds(b, _EB)], mb, s2),
                pltpu.make_async_copy(
                    dr_hbm.at[pl.ds(b // 128, _EB // 128)], drb, s3))

    def issue(t, efb, mb, drb, s1, s2, s3):
        for cp in copies(t, efb, mb, drb, s1, s2, s3):
            cp.start()

    def compute(t, efb, mb, drb, s1, s2, s3):
        for cp in copies(t, efb, mb, drb, s1, s2, s3):
            cp.wait()
        b = base0 + t * _EB
        gate = jax.nn.sigmoid(
            jnp.dot(efb[...].astype(jnp.bfloat16), wg,
                    preferred_element_type=_F32) + bg)
        val = gate * _unpack_bf16(mb[...]).astype(_F32)
        gidx = lax.broadcasted_iota(jnp.int32, (_EB, 1), 0) + b
        keep = (gidx >= s) & (gidx < e)
        rhs = jnp.where(keep, jnp.concatenate([val, gate], axis=1), 0.0)
        onehot = (drb[...][:, :, None]
                  == lax.broadcasted_iota(jnp.int32, (_EB // 128, 128, _WC), 2))
        onehot = onehot.astype(jnp.bfloat16).reshape(_EB, _WC)
        acc[...] += lax.dot_general(
            onehot, rhs.astype(jnp.bfloat16), (((0,), (0,)), ((), ())),
            preferred_element_type=_F32)

    buf_a = (ef_a, m_a, dr_a, sa1, sa2, sa3)
    buf_b = (ef_b, m_b, dr_b, sb1, sb2, sb3)

    @pl.when(n_it > 0)
    def _():
        issue(0, *buf_a)

    def it(t, carry):
        @pl.when(lax.rem(t, 2) == 0)
        def _():
            @pl.when(t + 1 < n_it)
            def _():
                issue(t + 1, *buf_b)
            compute(t, *buf_a)

        @pl.when(lax.rem(t, 2) == 1)
        def _():
            @pl.when(t + 1 < n_it)
            def _():
                issue(t + 1, *buf_a)
            compute(t, *buf_b)

        return carry

    lax.fori_loop(0, n_it, it, 0)

    h = h_ref[...]
    ad = acc[...]
    agg = ad[:, :gh]
    den = ad[:, gh:]
    x = (jnp.dot(h, ws_ref[...], preferred_element_type=_F32) + bs_ref[...]
         + agg / (den + 1e-6))
    h2 = _ln(_silu(x) + h, g_ref[...], b_ref[...])
    h2_ref[...] = h2
    m2_ref[...] = _pack_bf16(
        jnp.dot(h2, wm_ref[...], preferred_element_type=_F32))
    hbf_ref[...] = _pack_bf16(h2)


def _seg_call(starts, ef, msrc, dr2d, wg, bg, h, ws, bs, g, b, wm):
    eh = ef.shape[1]
    gh = wg.shape[1]
    nc = starts.shape[0] - 1
    grid = (nc,)
    full = lambda c: (0, 0)
    return pl.pallas_call(
        functools.partial(_seg_body, eh=eh, gh=gh),
        grid=grid,
        in_specs=[
            pl.BlockSpec(memory_space=pltpu.SMEM),
            pl.BlockSpec(memory_space=pl.ANY),
            pl.BlockSpec(memory_space=pl.ANY),
            pl.BlockSpec(memory_space=pl.ANY),
            pl.BlockSpec(wg.shape, full),
            pl.BlockSpec(bg.shape, full),
            pl.BlockSpec((_WC, gh), lambda c: (c, 0)),
            pl.BlockSpec(ws.shape, full),
            pl.BlockSpec(bs.shape, full),
            pl.BlockSpec(g.shape, full),
            pl.BlockSpec(b.shape, full),
            pl.BlockSpec(wm.shape, full),
        ],
        out_specs=[
            pl.BlockSpec((_WC, gh), lambda c: (c, 0)),
            pl.BlockSpec((_WC, gh // 2), lambda c: (c, 0)),
            pl.BlockSpec((_WC, gh // 2), lambda c: (c, 0)),
        ],
        out_shape=[
            jax.ShapeDtypeStruct((nc * _WC, gh), _F32),
            jax.ShapeDtypeStruct((nc * _WC, gh // 2), _F32),
            jax.ShapeDtypeStruct((nc * _WC, gh // 2), _F32),
        ],
        scratch_shapes=[
            pltpu.VMEM((_WC, 2 * gh), _F32),
            pltpu.VMEM((_EB, eh), _F32),
            pltpu.VMEM((_EB, gh // 2), _F32),
            pltpu.VMEM((_EB // 128, 128), jnp.int32),
            pltpu.VMEM((_EB, eh), _F32),
            pltpu.VMEM((_EB, gh // 2), _F32),
            pltpu.VMEM((_EB // 128, 128), jnp.int32),
            pltpu.SemaphoreType.DMA,
            pltpu.SemaphoreType.DMA,
            pltpu.SemaphoreType.DMA,
            pltpu.SemaphoreType.DMA,
            pltpu.SemaphoreType.DMA,
            pltpu.SemaphoreType.DMA,
        ],
    )(starts, ef, msrc, dr2d, wg, bg, h, ws, bs, g, b, wm)


def _head_body(ef_ref, hs_ref, hd_ref, w1e_ref, w1s_ref, w1d_ref, b1_ref,
               w2_ref, b2_ref, w3_ref, b3_ref, wo_ref, bo_ref, out_ref):
    bf = jnp.bfloat16
    hs = _unpack_bf16(hs_ref[...])
    hd = _unpack_bf16(hd_ref[...])
    p = _silu(jnp.dot(ef_ref[...].astype(bf), w1e_ref[...].astype(bf),
                      preferred_element_type=_F32)
              + jnp.dot(hs, w1s_ref[...].astype(bf),
                        preferred_element_type=_F32)
              + jnp.dot(hd, w1d_ref[...].astype(bf),
                        preferred_element_type=_F32)
              + b1_ref[...])
    p = _silu(jnp.dot(p.astype(bf), w2_ref[...].astype(bf),
                      preferred_element_type=_F32) + b2_ref[...])
    p = _silu(jnp.dot(p.astype(bf), w3_ref[...].astype(bf),
                      preferred_element_type=_F32) + b3_ref[...])
    out_ref[...] = jnp.dot(p, wo_ref[...], preferred_element_type=_F32) + bo_ref[...]


def _head_call(ef, hsrc, hdst, w1e, w1s, w1d, b1, w2, b2, w3, b3, wo, bo):
    ew = ef.shape[0]
    grid = (ew // _BE,)
    full = lambda i: (0, 0)
    return pl.pallas_call(
        _head_body,
        grid=grid,
        in_specs=[
            pl.BlockSpec((_BE, ef.shape[1]), lambda i: (i, 0)),
            pl.BlockSpec((_BE, hsrc.shape[1]), lambda i: (i, 0)),
            pl.BlockSpec((_BE, hdst.shape[1]), lambda i: (i, 0)),
            pl.BlockSpec(w1e.shape, full),
            pl.BlockSpec(w1s.shape, full),
            pl.BlockSpec(w1d.shape, full),
            pl.BlockSpec(b1.shape, full),
            pl.BlockSpec(w2.shape, full),
            pl.BlockSpec(b2.shape, full),
            pl.BlockSpec(w3.shape, full),
            pl.BlockSpec(b3.shape, full),
            pl.BlockSpec(wo.shape, full),
            pl.BlockSpec(bo.shape, full),
        ],
        out_specs=pl.BlockSpec((_BE, 16), lambda i: (i, 0)),
        out_shape=jax.ShapeDtypeStruct((ew, 16), _F32),
    )(ef, hsrc, hdst, w1e, w1s, w1d, b1, w2, b2, w3, b3, wo, bo)


def kernel(atom_features, edge_index, r, params):
    n, af = atom_features.shape
    e = edge_index.shape[1]
    nl = params["Wm"].shape[0]
    gh = params["Wm"].shape[2]
    eh = params["W_e2"].shape[0]
    bins = params["W_e1"].shape[0] - 2 * af

    src = edge_index[0]
    dst = edge_index[1]

    eidx = jnp.arange(e, dtype=jnp.int32)
    dst_s, perm, src_s = lax.sort((dst, eidx, src), num_keys=1)

    chunk = _NW * _CH * 4
    ew = ((e + _EB + chunk - 1) // chunk) * chunk
    padw = ew - e

    def padi(x):
        return jnp.concatenate([x, jnp.zeros((padw,), jnp.int32)])

    perm_p = padi(perm)
    srcs_p = padi(src_s)
    src_p = padi(src)
    dst_p = padi(dst)
    drel = padi((dst_s % _WC).astype(jnp.int32)).reshape(ew // 128, 128)

    nc = (n + _WC - 1) // _WC
    bounds = (jnp.arange(nc + 1, dtype=jnp.int32) * _WC).astype(dst_s.dtype)
    starts = jnp.searchsorted(dst_s, bounds).astype(jnp.int32)

    afp = ((af + 127) // 128) * 128
    af_pad = afp - af
    afx = jnp.pad(atom_features, ((0, 0), (0, af_pad)))
    w_emb = jnp.pad(params["W_emb"], ((0, af_pad), (0, 0)))
    dist = jnp.sqrt(jnp.sum(r * r, axis=1))
    d2d = jnp.concatenate([dist, jnp.zeros((padw,), _F32)]).reshape(
        ew // 128, 128)

    apk = _pack_bf16(jnp.pad(atom_features, ((0, 0), (0, 256 - af))))
    w_e1 = params["W_e1"]
    w1a = jnp.pad(w_e1[:af], ((0, 256 - af), (0, 0)))
    w1b = jnp.pad(w_e1[af:2 * af], ((0, 256 - af), (0, 0)))
    w1r = w_e1[2 * af:]
    centers = jnp.linspace(0.0, 8.0, bins, dtype=_F32).reshape(1, bins)
    width = 8.0 / bins

    row = lambda v: v.reshape(1, -1)

    wp1 = params["Wp1"]
    w1e = wp1[:eh]
    w1s = wp1[eh:eh + gh]
    w1d = wp1[eh + gh:]
    wo = jnp.pad(params["Wo"], ((0, 0), (0, 16 - params["Wo"].shape[1])))
    bo = jnp.pad(params["bo"], (0, 16 - params["bo"].shape[0])).reshape(1, 16)

    h, m = _embed_call(afx, w_emb, row(params["b_emb"]), row(params["g_emb"]),
                       row(params["be_emb"]), params["Wm"][0])

    asrc, adst = _sc_gather2(apk, src_p, dst_p)

    ef = _enc_call(asrc, adst, d2d, centers, w1a, w1b, w1r,
                   row(params["b_e1"]), params["W_e2"], row(params["b_e2"]),
                   row(params["g_e"]), row(params["be_e"]), width * width)

    ef_s = _sc_gather(ef, perm_p)

    hbf = None
    for i in range(nl):
        msrc = _sc_gather(m, srcs_p)
        wm_next = params["Wm"][(i + 1) % nl]
        h, m, hbf = _seg_call(starts, ef_s, msrc, drel,
                              params["Wg"][i], row(params["bg"][i]),
                              h, params["Ws"][i], row(params["bs"][i]),
                              row(params["g_ln"][i]), row(params["b_ln"][i]),
                              wm_next)

    hsrc, hdst = _sc_gather2(hbf, src_p, dst_p)

    outp = _head_call(ef, hsrc, hdst, w1e, w1s, w1d, row(params["bp1"]),
                      params["Wp2"], row(params["bp2"]),
                      params["Wp3"], row(params["bp3"]), wo, bo)

    return outp[:e, :1]


# speedup vs baseline: 1.3186x; 1.3186x over previous
import functools

import jax
import jax.numpy as jnp
from jax import lax
from jax.experimental import pallas as pl
from jax.experimental.pallas import tpu as pltpu
from jax.experimental.pallas import tpu_sc as plsc

_SC_CORES = 2
_SC_SUBCORES = 16
_NW = _SC_CORES * _SC_SUBCORES
_CH = 128
_WC = 128
_EB = 2048
_BN = 1000
_BE = 2048

_F32 = jnp.float32


def _ln(x, g, b):
    mu = jnp.mean(x, axis=-1, keepdims=True)
    var = jnp.mean((x - mu) ** 2, axis=-1, keepdims=True)
    return g * (x - mu) / jnp.sqrt(var + 1e-5) + b


def _silu(x):
    return x * jax.nn.sigmoid(x)


def _pack_bf16(x):
    k = x.shape[-1] // 2
    a = x[:, :k].astype(jnp.bfloat16)
    b = x[:, k:].astype(jnp.bfloat16)
    au = lax.bitcast_convert_type(a, jnp.uint16).astype(jnp.uint32)
    bu = lax.bitcast_convert_type(b, jnp.uint16).astype(jnp.uint32)
    return lax.bitcast_convert_type((au << 16) | bu, _F32)


def _unpack_bf16(p):
    u = lax.bitcast_convert_type(p, jnp.uint32)
    hi = lax.bitcast_convert_type((u >> 16).astype(jnp.uint16), jnp.bfloat16)
    lo = lax.bitcast_convert_type(u.astype(jnp.uint16), jnp.bfloat16)
    return jnp.concatenate([hi, lo], axis=-1)


def _sc_gather(table, idx):
    ew = idx.shape[0]
    tail = table.shape[1:]
    per_w = ew // _NW
    nring = 4
    n4 = per_w // (nring * _CH)
    mesh = plsc.VectorSubcoreMesh(core_axis_name="c", subcore_axis_name="s")

    @functools.partial(
        pl.kernel,
        out_type=jax.ShapeDtypeStruct((ew,) + tail, table.dtype),
        mesh=mesh,
        scratch_types=[
            pltpu.VMEM((per_w,), jnp.int32),
        ] + [pltpu.VMEM((_CH,) + tail, table.dtype)] * nring
          + [pltpu.SemaphoreType.DMA] * (2 * nring),
    )
    def gk(table_hbm, idx_hbm, out_hbm, idx_v, *bufs_sems):
        rows = bufs_sems[:nring]
        sems = bufs_sems[nring:2 * nring]
        wsems = bufs_sems[2 * nring:]
        wid = lax.axis_index("s") * _SC_CORES + lax.axis_index("c")
        base = wid * per_w
        pltpu.sync_copy(idx_hbm.at[pl.ds(base, per_w)], idx_v)

        def body(p, carry):
            offs = [(nring * p + j) * _CH for j in range(nring)]
            descs = []
            for j in range(nring):
                @pl.when(p > 0)
                def _(j=j):
                    pltpu.make_async_copy(
                        rows[j], out_hbm.at[pl.ds(base + offs[j], _CH)],
                        wsems[j]).wait()
                descs.append(pltpu.async_copy(
                    table_hbm.at[idx_v.at[pl.ds(offs[j], _CH)]],
                    rows[j], sems[j]))
            for j in range(nring):
                descs[j].wait()
                pltpu.async_copy(
                    rows[j], out_hbm.at[pl.ds(base + offs[j], _CH)],
                    wsems[j])
            return carry

        lax.fori_loop(0, n4, body, 0)
        for j in range(nring):
            pltpu.make_async_copy(
                rows[j], out_hbm.at[pl.ds(base, _CH)], wsems[j]).wait()

    return gk(table, idx)


def _sc_gather2(table, idx_a, idx_b, table_b=None):
    ew = idx_a.shape[0]
    d = table.shape[1]
    per_w = ew // _NW
    n_ch = per_w // _CH
    mesh = plsc.VectorSubcoreMesh(core_axis_name="c", subcore_axis_name="s")

    @functools.partial(
        pl.kernel,
        out_type=[
            jax.ShapeDtypeStruct((ew, d), _F32),
            jax.ShapeDtypeStruct((ew, d), _F32),
        ],
        mesh=mesh,
        scratch_types=[
            pltpu.VMEM((per_w,), jnp.int32),
            pltpu.VMEM((per_w,), jnp.int32),
            pltpu.VMEM((_CH, d), _F32),
            pltpu.VMEM((_CH, d), _F32),
            pltpu.SemaphoreType.DMA,
            pltpu.SemaphoreType.DMA,
            pltpu.SemaphoreType.DMA,
            pltpu.SemaphoreType.DMA,
        ],
    )
    def gk(table_hbm, tb_hbm, ia_hbm, ib_hbm, oa_hbm, ob_hbm,
           ia_v, ib_v, rows0, rows1, sem0, sem1, wsem0, wsem1):
        wid = lax.axis_index("s") * _SC_CORES + lax.axis_index("c")
        base = wid * per_w
        pltpu.sync_copy(ia_hbm.at[pl.ds(base, per_w)], ia_v)
        pltpu.sync_copy(ib_hbm.at[pl.ds(base, per_w)], ib_v)

        def body(t, carry):
            off = t * _CH

            @pl.when(t > 0)
            def _():
                pltpu.make_async_copy(
                    rows0, oa_hbm.at[pl.ds(base + off, _CH)], wsem0).wait()
                pltpu.make_async_copy(
                    rows1, ob_hbm.at[pl.ds(base + off, _CH)], wsem1).wait()

            d0 = pltpu.async_copy(
                table_hbm.at[ia_v.at[pl.ds(off, _CH)]], rows0, sem0)
            d1 = pltpu.async_copy(
                tb_hbm.at[ib_v.at[pl.ds(off, _CH)]], rows1, sem1)
            d0.wait()
            pltpu.async_copy(rows0, oa_hbm.at[pl.ds(base + off, _CH)], wsem0)
            d1.wait()
            pltpu.async_copy(rows1, ob_hbm.at[pl.ds(base + off, _CH)], wsem1)
            return carry

        lax.fori_loop(0, n_ch, body, 0)
        pltpu.make_async_copy(
            rows0, oa_hbm.at[pl.ds(base, _CH)], wsem0).wait()
        pltpu.make_async_copy(
            rows1, ob_hbm.at[pl.ds(base, _CH)], wsem1).wait()

    return gk(table, table_b if table_b is not None else table, idx_a, idx_b)


def _embed_body(af_ref, w_ref, b_ref, g_ref, be_ref, wm_ref, h_ref, m_ref):
    x = jnp.dot(af_ref[...], w_ref[...], preferred_element_type=_F32) + b_ref[...]
    h = _ln(_silu(x), g_ref[...], be_ref[...])
    h_ref[...] = h
    m_ref[...] = _pack_bf16(
        jnp.dot(h, wm_ref[...], preferred_element_type=_F32))


def _embed_call(af, w, b, g, be, wm):
    n = af.shape[0]
    gh = w.shape[1]
    grid = (n // _BN,)
    full = lambda i: (0, 0)
    return pl.pallas_call(
        _embed_body,
        grid=grid,
        in_specs=[
            pl.BlockSpec((_BN, af.shape[1]), lambda i: (i, 0)),
            pl.BlockSpec(w.shape, full),
            pl.BlockSpec(b.shape, full),
            pl.BlockSpec(g.shape, full),
            pl.BlockSpec(be.shape, full),
            pl.BlockSpec(wm.shape, full),
        ],
        out_specs=[
            pl.BlockSpec((_BN, gh), lambda i: (i, 0)),
            pl.BlockSpec((_BN, gh // 2), lambda i: (i, 0)),
        ],
        out_shape=[
            jax.ShapeDtypeStruct((n, gh), _F32),
            jax.ShapeDtypeStruct((n, gh // 2), _F32),
        ],
    )(af, w, b, g, be, wm)


def _enc_body(asrc_ref, adst_ref, d_ref, c_ref, w1a_ref, w1b_ref, w1r_ref,
              b1_ref, w2_ref, b2_ref, g_ref, be_ref, out_ref, *, wsq):
    bf = jnp.bfloat16
    dd = d_ref[...]
    nb = dd.shape[0]
    bins = c_ref.shape[1]
    dist3 = dd[:, :, None]
    cen3 = c_ref[...].reshape(1, 1, bins)
    rbf = jnp.exp(-((dist3 - cen3) ** 2) / wsq).reshape(nb * 128, bins)
    x = (jnp.dot(_unpack_bf16(asrc_ref[...]), w1a_ref[...].astype(bf),
                 preferred_element_type=_F32)
         + jnp.dot(_unpack_bf16(adst_ref[...]), w1b_ref[...].astype(bf),
                   preferred_element_type=_F32)
         + jnp.dot(rbf, w1r_ref[...], preferred_element_type=_F32)
         + b1_ref[...])
    x = _silu(x)
    y = _silu(jnp.dot(x.astype(bf), w2_ref[...].astype(bf),
                      preferred_element_type=_F32) + b2_ref[...])
    out_ref[...] = _ln(y, g_ref[...], be_ref[...])


def _enc_call(asrc, adst, d2d, centers, w1a, w1b, w1r, b1, w2, b2, g, be, wsq):
    ew = asrc.shape[0]
    eh = w2.shape[1]
    grid = (ew // _BE,)
    full = lambda i: (0, 0)
    return pl.pallas_call(
        functools.partial(_enc_body, wsq=wsq),
        grid=grid,
        in_specs=[
            pl.BlockSpec((_BE, asrc.shape[1]), lambda i: (i, 0)),
            pl.BlockSpec((_BE, adst.shape[1]), lambda i: (i, 0)),
            pl.BlockSpec((_BE // 128, 128), lambda i: (i, 0)),
            pl.BlockSpec(centers.shape, full),
            pl.BlockSpec(w1a.shape, full),
            pl.BlockSpec(w1b.shape, full),
            pl.BlockSpec(w1r.shape, full),
            pl.BlockSpec(b1.shape, full),
            pl.BlockSpec(w2.shape, full),
            pl.BlockSpec(b2.shape, full),
            pl.BlockSpec(g.shape, full),
            pl.BlockSpec(be.shape, full),
        ],
        out_specs=pl.BlockSpec((_BE, eh), lambda i: (i, 0)),
        out_shape=jax.ShapeDtypeStruct((ew, eh), _F32),
    )(asrc, adst, d2d, centers, w1a, w1b, w1r, b1, w2, b2, g, be)


def _seg_body(starts_ref, ef_hbm, m_hbm, dr_hbm, wg_ref, bg_ref,
              h_ref, ws_ref, bs_ref, g_ref, b_ref, wm_ref,
              h2_ref, m2_ref, hbf_ref,
              acc, ef_a, m_a, dr_a, ef_b, m_b, dr_b,
              sa1, sa2, sa3, sb1, sb2, sb3, *, eh, gh):
    c = pl.program_id(0)
    s = starts_ref[c]
    e = starts_ref[c + 1]
    base0 = (s // 128) * 128
    n_it = (e - base0 + _EB - 1) // _EB
    acc[...] = jnp.zeros_like(acc)
    wg = wg_ref[...].astype(jnp.bfloat16)
    bg = bg_ref[...]

    def copies(t, efb, mb, drb, s1, s2, s3):
        b = base0 + t * _EB
        return (pltpu.make_async_copy(ef_hbm.at[pl.ds(b, _EB)], efb, s1),
                pltpu.make_async_copy(m_hbm.at[pl.ds(b, _EB)], mb, s2),
                pltpu.make_async_copy(
                    dr_hbm.at[pl.ds(b // 128, _EB // 128)], drb, s3))

    def issue(t, efb, mb, drb, s1, s2, s3):
        for cp in copies(t, efb, mb, drb, s1, s2, s3):
            cp.start()

    def compute(t, efb, mb, drb, s1, s2, s3):
        for cp in copies(t, efb, mb, drb, s1, s2, s3):
            cp.wait()
        b = base0 + t * _EB
        gate = jax.nn.sigmoid(
            jnp.dot(efb[...].astype(jnp.bfloat16), wg,
                    preferred_element_type=_F32) + bg)
        val = gate * _unpack_bf16(mb[...]).astype(_F32)
        gidx = lax.broadcasted_iota(jnp.int32, (_EB, 1), 0) + b
        keep = (gidx >= s) & (gidx < e)
        rhs = jnp.where(keep, jnp.concatenate([val, gate], axis=1), 0.0)
        onehot = (drb[...][:, :, None]
                  == lax.broadcasted_iota(jnp.int32, (_EB // 128, 128, _WC), 2))
        onehot = onehot.astype(jnp.bfloat16).reshape(_EB, _WC)
        acc[...] += lax.dot_general(
            onehot, rhs.astype(jnp.bfloat16), (((0,), (0,)), ((), ())),
            preferred_element_type=_F32)

    buf_a = (ef_a, m_a, dr_a, sa1, sa2, sa3)
    buf_b = (ef_b, m_b, dr_b, sb1, sb2, sb3)

    @pl.when(n_it > 0)
    def _():
        issue(0, *buf_a)

    def it(t, carry):
        @pl.when(lax.rem(t, 2) == 0)
        def _():
            @pl.when(t + 1 < n_it)
            def _():
                issue(t + 1, *buf_b)
            compute(t, *buf_a)

        @pl.when(lax.rem(t, 2) == 1)
        def _():
            @pl.when(t + 1 < n_it)
            def _():
                issue(t + 1, *buf_a)
            compute(t, *buf_b)

        return carry

    lax.fori_loop(0, n_it, it, 0)

    h = h_ref[...]
    ad = acc[...]
    agg = ad[:, :gh]
    den = ad[:, gh:]
    x = (jnp.dot(h, ws_ref[...], preferred_element_type=_F32) + bs_ref[...]
         + agg / (den + 1e-6))
    h2 = _ln(_silu(x) + h, g_ref[...], b_ref[...])
    h2_ref[...] = h2
    m2_ref[...] = _pack_bf16(
        jnp.dot(h2, wm_ref[...], preferred_element_type=_F32))
    hbf_ref[...] = _pack_bf16(h2)


def _seg_call(starts, ef, msrc, dr2d, wg, bg, h, ws, bs, g, b, wm):
    eh = ef.shape[1]
    gh = wg.shape[1]
    nc = starts.shape[0] - 1
    grid = (nc,)
    full = lambda c: (0, 0)
    return pl.pallas_call(
        functools.partial(_seg_body, eh=eh, gh=gh),
        grid=grid,
        in_specs=[
            pl.BlockSpec(memory_space=pltpu.SMEM),
            pl.BlockSpec(memory_space=pl.ANY),
            pl.BlockSpec(memory_space=pl.ANY),
            pl.BlockSpec(memory_space=pl.ANY),
            pl.BlockSpec(wg.shape, full),
            pl.BlockSpec(bg.shape, full),
            pl.BlockSpec((_WC, gh), lambda c: (c, 0)),
            pl.BlockSpec(ws.shape, full),
            pl.BlockSpec(bs.shape, full),
            pl.BlockSpec(g.shape, full),
            pl.BlockSpec(b.shape, full),
            pl.BlockSpec(wm.shape, full),
        ],
        out_specs=[
            pl.BlockSpec((_WC, gh), lambda c: (c, 0)),
            pl.BlockSpec((_WC, gh // 2), lambda c: (c, 0)),
            pl.BlockSpec((_WC, gh // 2), lambda c: (c, 0)),
        ],
        out_shape=[
            jax.ShapeDtypeStruct((nc * _WC, gh), _F32),
            jax.ShapeDtypeStruct((nc * _WC, gh // 2), _F32),
            jax.ShapeDtypeStruct((nc * _WC, gh // 2), _F32),
        ],
        scratch_shapes=[
            pltpu.VMEM((_WC, 2 * gh), _F32),
            pltpu.VMEM((_EB, eh), _F32),
            pltpu.VMEM((_EB, gh // 2), _F32),
            pltpu.VMEM((_EB // 128, 128), jnp.int32),
            pltpu.VMEM((_EB, eh), _F32),
            pltpu.VMEM((_EB, gh // 2), _F32),
            pltpu.VMEM((_EB // 128, 128), jnp.int32),
            pltpu.SemaphoreType.DMA,
            pltpu.SemaphoreType.DMA,
            pltpu.SemaphoreType.DMA,
            pltpu.SemaphoreType.DMA,
            pltpu.SemaphoreType.DMA,
            pltpu.SemaphoreType.DMA,
        ],
    )(starts, ef, msrc, dr2d, wg, bg, h, ws, bs, g, b, wm)


def _head_body(ef_ref, hs_ref, hd_ref, w1e_ref, w1s_ref, w1d_ref, b1_ref,
               w2_ref, b2_ref, w3_ref, b3_ref, wo_ref, bo_ref, out_ref):
    bf = jnp.bfloat16
    hs = _unpack_bf16(hs_ref[...])
    hd = _unpack_bf16(hd_ref[...])
    p = _silu(jnp.dot(ef_ref[...].astype(bf), w1e_ref[...].astype(bf),
                      preferred_element_type=_F32)
              + jnp.dot(hs, w1s_ref[...].astype(bf),
                        preferred_element_type=_F32)
              + jnp.dot(hd, w1d_ref[...].astype(bf),
                        preferred_element_type=_F32)
              + b1_ref[...])
    p = _silu(jnp.dot(p.astype(bf), w2_ref[...].astype(bf),
                      preferred_element_type=_F32) + b2_ref[...])
    p = _silu(jnp.dot(p.astype(bf), w3_ref[...].astype(bf),
                      preferred_element_type=_F32) + b3_ref[...])
    out_ref[...] = jnp.dot(p, wo_ref[...], preferred_element_type=_F32) + bo_ref[...]


def _head_call(ef, hsrc, hdst, w1e, w1s, w1d, b1, w2, b2, w3, b3, wo, bo):
    ew = ef.shape[0]
    grid = (ew // _BE,)
    full = lambda i: (0, 0)
    return pl.pallas_call(
        _head_body,
        grid=grid,
        in_specs=[
            pl.BlockSpec((_BE, ef.shape[1]), lambda i: (i, 0)),
            pl.BlockSpec((_BE, hsrc.shape[1]), lambda i: (i, 0)),
            pl.BlockSpec((_BE, hdst.shape[1]), lambda i: (i, 0)),
            pl.BlockSpec(w1e.shape, full),
            pl.BlockSpec(w1s.shape, full),
            pl.BlockSpec(w1d.shape, full),
            pl.BlockSpec(b1.shape, full),
            pl.BlockSpec(w2.shape, full),
            pl.BlockSpec(b2.shape, full),
            pl.BlockSpec(w3.shape, full),
            pl.BlockSpec(b3.shape, full),
            pl.BlockSpec(wo.shape, full),
            pl.BlockSpec(bo.shape, full),
        ],
        out_specs=pl.BlockSpec((_BE, 16), lambda i: (i, 0)),
        out_shape=jax.ShapeDtypeStruct((ew, 16), _F32),
    )(ef, hsrc, hdst, w1e, w1s, w1d, b1, w2, b2, w3, b3, wo, bo)


def kernel(atom_features, edge_index, r, params):
    n, af = atom_features.shape
    e = edge_index.shape[1]
    nl = params["Wm"].shape[0]
    gh = params["Wm"].shape[2]
    eh = params["W_e2"].shape[0]
    bins = params["W_e1"].shape[0] - 2 * af

    src = edge_index[0]
    dst = edge_index[1]

    eidx = jnp.arange(e, dtype=jnp.int32)
    dst_s, perm, src_s = lax.sort((dst, eidx, src), num_keys=1)

    chunk = _NW * _CH * 4
    ew = ((e + _EB + chunk - 1) // chunk) * chunk
    padw = ew - e

    def padi(x):
        return jnp.concatenate([x, jnp.zeros((padw,), jnp.int32)])

    perm_p = padi(perm)
    srcs_p = padi(src_s)
    src_p = padi(src)
    dst_p = padi(dst)
    drel = padi((dst_s % _WC).astype(jnp.int32)).reshape(ew // 128, 128)

    nc = (n + _WC - 1) // _WC
    bounds = (jnp.arange(nc + 1, dtype=jnp.int32) * _WC).astype(dst_s.dtype)
    starts = jnp.searchsorted(dst_s, bounds).astype(jnp.int32)

    afp = ((af + 127) // 128) * 128
    af_pad = afp - af
    afx = jnp.pad(atom_features, ((0, 0), (0, af_pad)))
    w_emb = jnp.pad(params["W_emb"], ((0, af_pad), (0, 0)))
    dist = jnp.sqrt(jnp.sum(r * r, axis=1))
    d2d = jnp.concatenate([dist, jnp.zeros((padw,), _F32)]).reshape(
        ew // 128, 128)

    apk = _pack_bf16(jnp.pad(atom_features, ((0, 0), (0, 256 - af))))
    w_e1 = params["W_e1"]
    w1a = jnp.pad(w_e1[:af], ((0, 256 - af), (0, 0)))
    w1b = jnp.pad(w_e1[af:2 * af], ((0, 256 - af), (0, 0)))
    w1r = w_e1[2 * af:]
    centers = jnp.linspace(0.0, 8.0, bins, dtype=_F32).reshape(1, bins)
    width = 8.0 / bins

    row = lambda v: v.reshape(1, -1)

    wp1 = params["Wp1"]
    w1e = wp1[:eh]
    w1s = wp1[eh:eh + gh]
    w1d = wp1[eh + gh:]
    wo = jnp.pad(params["Wo"], ((0, 0), (0, 16 - params["Wo"].shape[1])))
    bo = jnp.pad(params["bo"], (0, 16 - params["bo"].shape[0])).reshape(1, 16)

    h, m = _embed_call(afx, w_emb, row(params["b_emb"]), row(params["g_emb"]),
                       row(params["be_emb"]), params["Wm"][0])

    asrc, adst = _sc_gather2(apk, src_p, dst_p)

    ef = _enc_call(asrc, adst, d2d, centers, w1a, w1b, w1r,
                   row(params["b_e1"]), params["W_e2"], row(params["b_e2"]),
                   row(params["g_e"]), row(params["be_e"]), width * width)

    ef_s = None
    hbf = None
    for i in range(nl):
        if i == 0:
            ef_s, msrc = _sc_gather2(ef, perm_p, srcs_p, table_b=m)
        else:
            msrc = _sc_gather(m, srcs_p)
        wm_next = params["Wm"][(i + 1) % nl]
        h, m, hbf = _seg_call(starts, ef_s, msrc, drel,
                              params["Wg"][i], row(params["bg"][i]),
                              h, params["Ws"][i], row(params["bs"][i]),
                              row(params["g_ln"][i]), row(params["b_ln"][i]),
                              wm_next)

    hsrc, hdst = _sc_gather2(hbf, src_p, dst_p)

    outp = _head_call(ef, hsrc, hdst, w1e, w1s, w1d, row(params["bp1"]),
                      params["Wp2"], row(params["bp2"]),
                      params["Wp3"], row(params["bp3"]), wo, bo)

    return outp[:e, :1]


# speedup vs baseline: 1.3804x; 1.0469x over previous
import functools

import jax
import jax.numpy as jnp
from jax import lax
from jax.experimental import pallas as pl
from jax.experimental.pallas import tpu as pltpu
from jax.experimental.pallas import tpu_sc as plsc

_SC_CORES = 2
_SC_SUBCORES = 16
_NW = _SC_CORES * _SC_SUBCORES
_CH = 128
_WC = 512
_EB = 2048
_BN = 1000
_BE = 4096

_F32 = jnp.float32


def _ln(x, g, b):
    mu = jnp.mean(x, axis=-1, keepdims=True)
    var = jnp.mean((x - mu) ** 2, axis=-1, keepdims=True)
    return g * (x - mu) / jnp.sqrt(var + 1e-5) + b


def _silu(x):
    return x * jax.nn.sigmoid(x)


def _pack_bf16(x):
    k = x.shape[-1] // 2
    a = x[:, :k].astype(jnp.bfloat16)
    b = x[:, k:].astype(jnp.bfloat16)
    au = lax.bitcast_convert_type(a, jnp.uint16).astype(jnp.uint32)
    bu = lax.bitcast_convert_type(b, jnp.uint16).astype(jnp.uint32)
    return lax.bitcast_convert_type((au << 16) | bu, _F32)


def _unpack_bf16(p):
    u = lax.bitcast_convert_type(p, jnp.uint32)
    hi = lax.bitcast_convert_type((u >> 16).astype(jnp.uint16), jnp.bfloat16)
    lo = lax.bitcast_convert_type(u.astype(jnp.uint16), jnp.bfloat16)
    return jnp.concatenate([hi, lo], axis=-1)


def _sc_gather(table, idx):
    ew = idx.shape[0]
    tail = table.shape[1:]
    per_w = ew // _NW
    nring = 4
    n4 = per_w // (nring * _CH)
    mesh = plsc.VectorSubcoreMesh(core_axis_name="c", subcore_axis_name="s")

    @functools.partial(
        pl.kernel,
        out_type=jax.ShapeDtypeStruct((ew,) + tail, table.dtype),
        mesh=mesh,
        scratch_types=[
            pltpu.VMEM((per_w,), jnp.int32),
        ] + [pltpu.VMEM((_CH,) + tail, table.dtype)] * nring
          + [pltpu.SemaphoreType.DMA] * (2 * nring),
    )
    def gk(table_hbm, idx_hbm, out_hbm, idx_v, *bufs_sems):
        rows = bufs_sems[:nring]
        sems = bufs_sems[nring:2 * nring]
        wsems = bufs_sems[2 * nring:]
        wid = lax.axis_index("s") * _SC_CORES + lax.axis_index("c")
        base = wid * per_w
        pltpu.sync_copy(idx_hbm.at[pl.ds(base, per_w)], idx_v)

        def body(p, carry):
            offs = [(nring * p + j) * _CH for j in range(nring)]
            descs = []
            for j in range(nring):
                @pl.when(p > 0)
                def _(j=j):
                    pltpu.make_async_copy(
                        rows[j], out_hbm.at[pl.ds(base + offs[j], _CH)],
                        wsems[j]).wait()
                descs.append(pltpu.async_copy(
                    table_hbm.at[idx_v.at[pl.ds(offs[j], _CH)]],
                    rows[j], sems[j]))
            for j in range(nring):
                descs[j].wait()
                pltpu.async_copy(
                    rows[j], out_hbm.at[pl.ds(base + offs[j], _CH)],
                    wsems[j])
            return carry

        lax.fori_loop(0, n4, body, 0)
        for j in range(nring):
            pltpu.make_async_copy(
                rows[j], out_hbm.at[pl.ds(base, _CH)], wsems[j]).wait()

    return gk(table, idx)


def _sc_gather2(table, idx_a, idx_b, table_b=None):
    ew = idx_a.shape[0]
    d = table.shape[1]
    per_w = ew // _NW
    n_ch = per_w // _CH
    mesh = plsc.VectorSubcoreMesh(core_axis_name="c", subcore_axis_name="s")

    @functools.partial(
        pl.kernel,
        out_type=[
            jax.ShapeDtypeStruct((ew, d), _F32),
            jax.ShapeDtypeStruct((ew, d), _F32),
        ],
        mesh=mesh,
        scratch_types=[
            pltpu.VMEM((per_w,), jnp.int32),
            pltpu.VMEM((per_w,), jnp.int32),
            pltpu.VMEM((_CH, d), _F32),
            pltpu.VMEM((_CH, d), _F32),
            pltpu.SemaphoreType.DMA,
            pltpu.SemaphoreType.DMA,
            pltpu.SemaphoreType.DMA,
            pltpu.SemaphoreType.DMA,
        ],
    )
    def gk(table_hbm, tb_hbm, ia_hbm, ib_hbm, oa_hbm, ob_hbm,
           ia_v, ib_v, rows0, rows1, sem0, sem1, wsem0, wsem1):
        wid = lax.axis_index("s") * _SC_CORES + lax.axis_index("c")
        base = wid * per_w
        pltpu.sync_copy(ia_hbm.at[pl.ds(base, per_w)], ia_v)
        pltpu.sync_copy(ib_hbm.at[pl.ds(base, per_w)], ib_v)

        def body(t, carry):
            off = t * _CH

            @pl.when(t > 0)
            def _():
                pltpu.make_async_copy(
                    rows0, oa_hbm.at[pl.ds(base + off, _CH)], wsem0).wait()
                pltpu.make_async_copy(
                    rows1, ob_hbm.at[pl.ds(base + off, _CH)], wsem1).wait()

            d0 = pltpu.async_copy(
                table_hbm.at[ia_v.at[pl.ds(off, _CH)]], rows0, sem0)
            d1 = pltpu.async_copy(
                tb_hbm.at[ib_v.at[pl.ds(off, _CH)]], rows1, sem1)
            d0.wait()
            pltpu.async_copy(rows0, oa_hbm.at[pl.ds(base + off, _CH)], wsem0)
            d1.wait()
            pltpu.async_copy(rows1, ob_hbm.at[pl.ds(base + off, _CH)], wsem1)
            return carry

        lax.fori_loop(0, n_ch, body, 0)
        pltpu.make_async_copy(
            rows0, oa_hbm.at[pl.ds(base, _CH)], wsem0).wait()
        pltpu.make_async_copy(
            rows1, ob_hbm.at[pl.ds(base, _CH)], wsem1).wait()

    return gk(table, table_b if table_b is not None else table, idx_a, idx_b)


def _embed_body(af_ref, w_ref, b_ref, g_ref, be_ref, wm_ref, h_ref, m_ref):
    x = jnp.dot(af_ref[...], w_ref[...], preferred_element_type=_F32) + b_ref[...]
    h = _ln(_silu(x), g_ref[...], be_ref[...])
    h_ref[...] = h
    m_ref[...] = _pack_bf16(
        jnp.dot(h, wm_ref[...], preferred_element_type=_F32))


def _embed_call(af, w, b, g, be, wm):
    n = af.shape[0]
    gh = w.shape[1]
    grid = (n // _BN,)
    full = lambda i: (0, 0)
    return pl.pallas_call(
        _embed_body,
        grid=grid,
        in_specs=[
            pl.BlockSpec((_BN, af.shape[1]), lambda i: (i, 0)),
            pl.BlockSpec(w.shape, full),
            pl.BlockSpec(b.shape, full),
            pl.BlockSpec(g.shape, full),
            pl.BlockSpec(be.shape, full),
            pl.BlockSpec(wm.shape, full),
        ],
        out_specs=[
            pl.BlockSpec((_BN, gh), lambda i: (i, 0)),
            pl.BlockSpec((_BN, gh // 2), lambda i: (i, 0)),
        ],
        out_shape=[
            jax.ShapeDtypeStruct((n, gh), _F32),
            jax.ShapeDtypeStruct((n, gh // 2), _F32),
        ],
    )(af, w, b, g, be, wm)


def _enc_body(asrc_ref, adst_ref, d_ref, c_ref, w1a_ref, w1b_ref, w1r_ref,
              b1_ref, w2_ref, b2_ref, g_ref, be_ref, out_ref, *, wsq):
    bf = jnp.bfloat16
    dd = d_ref[...]
    nb = dd.shape[0]
    bins = c_ref.shape[1]
    dist3 = dd[:, :, None]
    cen3 = c_ref[...].reshape(1, 1, bins)
    rbf = jnp.exp(-((dist3 - cen3) ** 2) / wsq).reshape(nb * 128, bins)
    x = (jnp.dot(_unpack_bf16(asrc_ref[...]), w1a_ref[...].astype(bf),
                 preferred_element_type=_F32)
         + jnp.dot(_unpack_bf16(adst_ref[...]), w1b_ref[...].astype(bf),
                   preferred_element_type=_F32)
         + jnp.dot(rbf, w1r_ref[...], preferred_element_type=_F32)
         + b1_ref[...])
    x = _silu(x)
    y = _silu(jnp.dot(x.astype(bf), w2_ref[...].astype(bf),
                      preferred_element_type=_F32) + b2_ref[...])
    out_ref[...] = _ln(y, g_ref[...], be_ref[...])


def _enc_call(asrc, adst, d2d, centers, w1a, w1b, w1r, b1, w2, b2, g, be, wsq):
    ew = asrc.shape[0]
    eh = w2.shape[1]
    grid = (ew // _BE,)
    full = lambda i: (0, 0)
    return pl.pallas_call(
        functools.partial(_enc_body, wsq=wsq),
        grid=grid,
        in_specs=[
            pl.BlockSpec((_BE, asrc.shape[1]), lambda i: (i, 0)),
            pl.BlockSpec((_BE, adst.shape[1]), lambda i: (i, 0)),
            pl.BlockSpec((_BE // 128, 128), lambda i: (i, 0)),
            pl.BlockSpec(centers.shape, full),
            pl.BlockSpec(w1a.shape, full),
            pl.BlockSpec(w1b.shape, full),
            pl.BlockSpec(w1r.shape, full),
            pl.BlockSpec(b1.shape, full),
            pl.BlockSpec(w2.shape, full),
            pl.BlockSpec(b2.shape, full),
            pl.BlockSpec(g.shape, full),
            pl.BlockSpec(be.shape, full),
        ],
        out_specs=pl.BlockSpec((_BE, eh), lambda i: (i, 0)),
        out_shape=jax.ShapeDtypeStruct((ew, eh), _F32),
    )(asrc, adst, d2d, centers, w1a, w1b, w1r, b1, w2, b2, g, be)


def _seg_body(starts_ref, ef_hbm, m_hbm, dr_hbm, wg_ref, bg_ref,
              h_ref, ws_ref, bs_ref, g_ref, b_ref, wm_ref,
              h2_ref, m2_ref, hbf_ref,
              acc, ef_a, m_a, dr_a, ef_b, m_b, dr_b,
              sa1, sa2, sa3, sb1, sb2, sb3, *, eh, gh):
    c = pl.program_id(0)
    s = starts_ref[c]
    e = starts_ref[c + 1]
    base0 = (s // 128) * 128
    n_it = (e - base0 + _EB - 1) // _EB
    acc[...] = jnp.zeros_like(acc)
    wg = wg_ref[...].astype(jnp.bfloat16)
    bg = bg_ref[...]

    def copies(t, efb, mb, drb, s1, s2, s3):
        b = base0 + t * _EB
        return (pltpu.make_async_copy(ef_hbm.at[pl.ds(b, _EB)], efb, s1),
                pltpu.make_async_copy(m_hbm.at[pl.ds(b, _EB)], mb, s2),
                pltpu.make_async_copy(
                    dr_hbm.at[pl.ds(b // 128, _EB // 128)], drb, s3))

    def issue(t, efb, mb, drb, s1, s2, s3):
        for cp in copies(t, efb, mb, drb, s1, s2, s3):
            cp.start()

    def compute(t, efb, mb, drb, s1, s2, s3):
        for cp in copies(t, efb, mb, drb, s1, s2, s3):
            cp.wait()
        b = base0 + t * _EB
        gate = jax.nn.sigmoid(
            jnp.dot(efb[...].astype(jnp.bfloat16), wg,
                    preferred_element_type=_F32) + bg)
        val = gate * _unpack_bf16(mb[...]).astype(_F32)
        gidx = lax.broadcasted_iota(jnp.int32, (_EB, 1), 0) + b
        keep = (gidx >= s) & (gidx < e)
        rhs = jnp.where(keep, jnp.concatenate([val, gate], axis=1), 0.0)
        onehot = (drb[...][:, :, None]
                  == lax.broadcasted_iota(jnp.int32, (_EB // 128, 128, _WC), 2))
        onehot = onehot.astype(jnp.bfloat16).reshape(_EB, _WC)
        acc[...] += lax.dot_general(
            onehot, rhs.astype(jnp.bfloat16), (((0,), (0,)), ((), ())),
            preferred_element_type=_F32)

    buf_a = (ef_a, m_a, dr_a, sa1, sa2, sa3)
    buf_b = (ef_b, m_b, dr_b, sb1, sb2, sb3)

    @pl.when(n_it > 0)
    def _():
        issue(0, *buf_a)

    def it(t, carry):
        @pl.when(lax.rem(t, 2) == 0)
        def _():
            @pl.when(t + 1 < n_it)
            def _():
                issue(t + 1, *buf_b)
            compute(t, *buf_a)

        @pl.when(lax.rem(t, 2) == 1)
        def _():
            @pl.when(t + 1 < n_it)
            def _():
                issue(t + 1, *buf_a)
            compute(t, *buf_b)

        return carry

    lax.fori_loop(0, n_it, it, 0)

    h = h_ref[...]
    ad = acc[...]
    agg = ad[:, :gh]
    den = ad[:, gh:]
    x = (jnp.dot(h, ws_ref[...], preferred_element_type=_F32) + bs_ref[...]
         + agg / (den + 1e-6))
    h2 = _ln(_silu(x) + h, g_ref[...], b_ref[...])
    h2_ref[...] = h2
    m2_ref[...] = _pack_bf16(
        jnp.dot(h2, wm_ref[...], preferred_element_type=_F32))
    hbf_ref[...] = _pack_bf16(h2)


def _seg_call(starts, ef, msrc, dr2d, wg, bg, h, ws, bs, g, b, wm):
    eh = ef.shape[1]
    gh = wg.shape[1]
    nc = starts.shape[0] - 1
    grid = (nc,)
    full = lambda c: (0, 0)
    return pl.pallas_call(
        functools.partial(_seg_body, eh=eh, gh=gh),
        grid=grid,
        in_specs=[
            pl.BlockSpec(memory_space=pltpu.SMEM),
            pl.BlockSpec(memory_space=pl.ANY),
            pl.BlockSpec(memory_space=pl.ANY),
            pl.BlockSpec(memory_space=pl.ANY),
            pl.BlockSpec(wg.shape, full),
            pl.BlockSpec(bg.shape, full),
            pl.BlockSpec((_WC, gh), lambda c: (c, 0)),
            pl.BlockSpec(ws.shape, full),
            pl.BlockSpec(bs.shape, full),
            pl.BlockSpec(g.shape, full),
            pl.BlockSpec(b.shape, full),
            pl.BlockSpec(wm.shape, full),
        ],
        out_specs=[
            pl.BlockSpec((_WC, gh), lambda c: (c, 0)),
            pl.BlockSpec((_WC, gh // 2), lambda c: (c, 0)),
            pl.BlockSpec((_WC, gh // 2), lambda c: (c, 0)),
        ],
        out_shape=[
            jax.ShapeDtypeStruct((nc * _WC, gh), _F32),
            jax.ShapeDtypeStruct((nc * _WC, gh // 2), _F32),
            jax.ShapeDtypeStruct((nc * _WC, gh // 2), _F32),
        ],
        scratch_shapes=[
            pltpu.VMEM((_WC, 2 * gh), _F32),
            pltpu.VMEM((_EB, eh), _F32),
            pltpu.VMEM((_EB, gh // 2), _F32),
            pltpu.VMEM((_EB // 128, 128), jnp.int32),
            pltpu.VMEM((_EB, eh), _F32),
            pltpu.VMEM((_EB, gh // 2), _F32),
            pltpu.VMEM((_EB // 128, 128), jnp.int32),
            pltpu.SemaphoreType.DMA,
            pltpu.SemaphoreType.DMA,
            pltpu.SemaphoreType.DMA,
            pltpu.SemaphoreType.DMA,
            pltpu.SemaphoreType.DMA,
            pltpu.SemaphoreType.DMA,
        ],
    )(starts, ef, msrc, dr2d, wg, bg, h, ws, bs, g, b, wm)


def _head_body(ef_ref, hs_ref, hd_ref, w1e_ref, w1s_ref, w1d_ref, b1_ref,
               w2_ref, b2_ref, w3_ref, b3_ref, wo_ref, bo_ref, out_ref):
    bf = jnp.bfloat16
    hs = _unpack_bf16(hs_ref[...])
    hd = _unpack_bf16(hd_ref[...])
    p = _silu(jnp.dot(ef_ref[...].astype(bf), w1e_ref[...].astype(bf),
                      preferred_element_type=_F32)
              + jnp.dot(hs, w1s_ref[...].astype(bf),
                        preferred_element_type=_F32)
              + jnp.dot(hd, w1d_ref[...].astype(bf),
                        preferred_element_type=_F32)
              + b1_ref[...])
    p = _silu(jnp.dot(p.astype(bf), w2_ref[...].astype(bf),
                      preferred_element_type=_F32) + b2_ref[...])
    p = _silu(jnp.dot(p.astype(bf), w3_ref[...].astype(bf),
                      preferred_element_type=_F32) + b3_ref[...])
    out_ref[...] = jnp.dot(p, wo_ref[...], preferred_element_type=_F32) + bo_ref[...]


def _head_call(ef, hsrc, hdst, w1e, w1s, w1d, b1, w2, b2, w3, b3, wo, bo):
    ew = ef.shape[0]
    grid = (ew // _BE,)
    full = lambda i: (0, 0)
    return pl.pallas_call(
        _head_body,
        grid=grid,
        in_specs=[
            pl.BlockSpec((_BE, ef.shape[1]), lambda i: (i, 0)),
            pl.BlockSpec((_BE, hsrc.shape[1]), lambda i: (i, 0)),
            pl.BlockSpec((_BE, hdst.shape[1]), lambda i: (i, 0)),
            pl.BlockSpec(w1e.shape, full),
            pl.BlockSpec(w1s.shape, full),
            pl.BlockSpec(w1d.shape, full),
            pl.BlockSpec(b1.shape, full),
            pl.BlockSpec(w2.shape, full),
            pl.BlockSpec(b2.shape, full),
            pl.BlockSpec(w3.shape, full),
            pl.BlockSpec(b3.shape, full),
            pl.BlockSpec(wo.shape, full),
            pl.BlockSpec(bo.shape, full),
        ],
        out_specs=pl.BlockSpec((_BE, 16), lambda i: (i, 0)),
        out_shape=jax.ShapeDtypeStruct((ew, 16), _F32),
    )(ef, hsrc, hdst, w1e, w1s, w1d, b1, w2, b2, w3, b3, wo, bo)


def kernel(atom_features, edge_index, r, params):
    n, af = atom_features.shape
    e = edge_index.shape[1]
    nl = params["Wm"].shape[0]
    gh = params["Wm"].shape[2]
    eh = params["W_e2"].shape[0]
    bins = params["W_e1"].shape[0] - 2 * af

    src = edge_index[0]
    dst = edge_index[1]

    eidx = jnp.arange(e, dtype=jnp.int32)
    dst_s, perm, src_s = lax.sort((dst, eidx, src), num_keys=1)

    chunk = _NW * _CH * 4
    ew = ((e + _EB + chunk - 1) // chunk) * chunk
    padw = ew - e

    def padi(x):
        return jnp.concatenate([x, jnp.zeros((padw,), jnp.int32)])

    perm_p = padi(perm)
    srcs_p = padi(src_s)
    src_p = padi(src)
    dst_p = padi(dst)
    drel = padi((dst_s % _WC).astype(jnp.int32)).reshape(ew // 128, 128)

    nc = (n + _WC - 1) // _WC
    bounds = (jnp.arange(nc + 1, dtype=jnp.int32) * _WC).astype(dst_s.dtype)
    starts = jnp.searchsorted(dst_s, bounds).astype(jnp.int32)

    afp = ((af + 127) // 128) * 128
    af_pad = afp - af
    afx = jnp.pad(atom_features, ((0, 0), (0, af_pad)))
    w_emb = jnp.pad(params["W_emb"], ((0, af_pad), (0, 0)))
    dist = jnp.sqrt(jnp.sum(r * r, axis=1))
    d2d = jnp.concatenate([dist, jnp.zeros((padw,), _F32)]).reshape(
        ew // 128, 128)

    apk = _pack_bf16(jnp.pad(atom_features, ((0, 0), (0, 256 - af))))
    w_e1 = params["W_e1"]
    w1a = jnp.pad(w_e1[:af], ((0, 256 - af), (0, 0)))
    w1b = jnp.pad(w_e1[af:2 * af], ((0, 256 - af), (0, 0)))
    w1r = w_e1[2 * af:]
    centers = jnp.linspace(0.0, 8.0, bins, dtype=_F32).reshape(1, bins)
    width = 8.0 / bins

    row = lambda v: v.reshape(1, -1)

    wp1 = params["Wp1"]
    w1e = wp1[:eh]
    w1s = wp1[eh:eh + gh]
    w1d = wp1[eh + gh:]
    wo = jnp.pad(params["Wo"], ((0, 0), (0, 16 - params["Wo"].shape[1])))
    bo = jnp.pad(params["bo"], (0, 16 - params["bo"].shape[0])).reshape(1, 16)

    h, m = _embed_call(afx, w_emb, row(params["b_emb"]), row(params["g_emb"]),
                       row(params["be_emb"]), params["Wm"][0])

    asrc, adst = _sc_gather2(apk, src_p, dst_p)

    ef = _enc_call(asrc, adst, d2d, centers, w1a, w1b, w1r,
                   row(params["b_e1"]), params["W_e2"], row(params["b_e2"]),
                   row(params["g_e"]), row(params["be_e"]), width * width)

    ef_s = None
    hbf = None
    for i in range(nl):
        if i == 0:
            ef_s, msrc = _sc_gather2(ef, perm_p, srcs_p, table_b=m)
        else:
            msrc = _sc_gather(m, srcs_p)
        wm_next = params["Wm"][(i + 1) % nl]
        h, m, hbf = _seg_call(starts, ef_s, msrc, drel,
                              params["Wg"][i], row(params["bg"][i]),
                              h, params["Ws"][i], row(params["bs"][i]),
                              row(params["g_ln"][i]), row(params["b_ln"][i]),
                              wm_next)

    hsrc, hdst = _sc_gather2(hbf, src_p, dst_p)

    outp = _head_call(ef, hsrc, hdst, w1e, w1s, w1d, row(params["bp1"]),
                      params["Wp2"], row(params["bp2"]),
                      params["Wp3"], row(params["bp3"]), wo, bo)

    return outp[:e, :1]
